# Initial kernel scaffold; baseline (speedup 1.0000x reference)
import jax
import jax.numpy as jnp
from jax.experimental import pallas as pl


def kernel(prototypes, feats, feats_org):
    raise NotImplementedError("write your pallas kernel here")


# speedup vs baseline: 1.3314x; 1.3314x over previous
import jax
import jax.numpy as jnp
from jax.experimental import pallas as pl
from jax.experimental.pallas import tpu as pltpu

_TEMP = 0.1
_TAU0 = 0.1
_NSHIFT = 5


def _l2n(x, eps=1e-8):
    n = jnp.maximum(jnp.sqrt(jnp.sum(x * x, axis=-1, keepdims=True)), eps)
    return x / n


def _shift_body(proto_ref, feats_ref, fo_ref, pout_ref, simout_ref, fon_ref):
    b = pl.program_id(0)
    K = proto_ref.shape[1]
    N = feats_ref.shape[1]

    @pl.when(b == 0)
    def _():
        fon_ref[...] = _l2n(fo_ref[...])

    f = feats_ref[0]
    fn = _l2n(f)
    p = proto_ref[0]

    dn_last = (((1,), (1,)), ((), ()))
    dn_mid = (((1,), (0,)), ((), ()))

    tau = jnp.full((K, 1), _TAU0, jnp.float32)
    sim = jax.lax.dot_general(_l2n(p), fn, dn_last,
                              preferred_element_type=jnp.float32)
    kiota = jax.lax.broadcasted_iota(jnp.int32, (K, N), 0)

    for it in range(_NSHIFT):
        z = sim / (_TEMP * tau)
        z = z - jnp.max(z, axis=1, keepdims=True)
        e = jnp.exp(z)
        w = e / jnp.sum(e, axis=1, keepdims=True)
        colmax = jnp.max(w, axis=0, keepdims=True)
        idx = jnp.min(jnp.where(w == colmax, kiota, K), axis=0, keepdims=True)
        mask = kiota == idx
        w2 = jnp.where(mask, w, 0.0)
        p = jax.lax.dot_general(w2, f, dn_mid,
                                preferred_element_type=jnp.float32)
        if it < _NSHIFT - 1:
            sim = jax.lax.dot_general(_l2n(p), fn, dn_last,
                                      preferred_element_type=jnp.float32)
            msum = jnp.sum(jnp.where(mask, sim, 0.0), axis=1)
            ms = jnp.sum(mask.astype(jnp.float32), axis=1)
            has = ms >= 1.0
            density = 1.0 - jnp.where(has, msum / jnp.where(has, ms, 1.0), 0.0)
            tau = jnp.clip(density, 1e-10, None)[:, None]

    pout_ref[0] = p
    simout_ref[0] = jax.lax.dot_general(_l2n(p), fon_ref[...], dn_last,
                                        preferred_element_type=jnp.float32)


def kernel(prototypes, feats, feats_org):
    B, K, D = prototypes.shape
    N = feats.shape[1]
    M = feats_org.shape[0]
    pout, simout = pl.pallas_call(
        _shift_body,
        grid=(B,),
        in_specs=[
            pl.BlockSpec((1, K, D), lambda b: (b, 0, 0)),
            pl.BlockSpec((1, N, D), lambda b: (b, 0, 0)),
            pl.BlockSpec((M, D), lambda b: (0, 0)),
        ],
        out_specs=[
            pl.BlockSpec((1, K, D), lambda b: (b, 0, 0)),
            pl.BlockSpec((1, K, M), lambda b: (b, 0, 0)),
        ],
        out_shape=[
            jax.ShapeDtypeStruct((B, K, D), jnp.float32),
            jax.ShapeDtypeStruct((B, K, M), jnp.float32),
        ],
        scratch_shapes=[pltpu.VMEM((M, D), jnp.float32)],
    )(prototypes, feats, feats_org)
    return pout.reshape(B * K, D), simout.reshape(B * K, M)


# speedup vs baseline: 3.5671x; 2.6792x over previous
import jax
import jax.numpy as jnp
from jax.experimental import pallas as pl
from jax.experimental.pallas import tpu as pltpu

_TEMP = 0.1
_TAU0 = 0.1
_NSHIFT = 5


def _l2n(x, eps=1e-8):
    n = jnp.maximum(jnp.sqrt(jnp.sum(x * x, axis=-1, keepdims=True)), eps)
    return x / n


_G = 4


def _shift_body(proto_ref, feats_ref, fo_ref, pout_ref, simout_ref, fon_ref):
    g = pl.program_id(0)
    G, K, D = proto_ref.shape
    N = feats_ref.shape[1]

    @pl.when(g == 0)
    def _():
        fon_ref[...] = _l2n(fo_ref[...])

    f = feats_ref[...]
    fn = _l2n(f)
    p = proto_ref[...]

    dn_last = (((1,), (1,)), ((), ()))
    dn_mid = (((1,), (0,)), ((), ()))

    def sim_all(pn):
        return jnp.stack(
            [jax.lax.dot_general(pn[i], fn[i], dn_last,
                                 preferred_element_type=jnp.float32)
             for i in range(G)], axis=0)

    tau = jnp.full((G, K, 1), _TAU0, jnp.float32)
    sim = sim_all(_l2n(p))
    kiota = jax.lax.broadcasted_iota(jnp.int32, (G, K, N), 1)

    for it in range(_NSHIFT):
        z = sim / (_TEMP * tau)
        z = z - jnp.max(z, axis=2, keepdims=True)
        e = jnp.exp(z)
        w = e / jnp.sum(e, axis=2, keepdims=True)
        colmax = jnp.max(w, axis=1, keepdims=True)
        idx = jnp.min(jnp.where(w == colmax, kiota, K), axis=1, keepdims=True)
        mask = kiota == idx
        w2 = jnp.where(mask, w, 0.0)
        p = jnp.stack(
            [jax.lax.dot_general(w2[i], f[i], dn_mid,
                                 preferred_element_type=jnp.float32)
             for i in range(G)], axis=0)
        if it < _NSHIFT - 1:
            sim = sim_all(_l2n(p))
            msum = jnp.sum(jnp.where(mask, sim, 0.0), axis=2)
            ms = jnp.sum(mask.astype(jnp.float32), axis=2)
            has = ms >= 1.0
            density = 1.0 - jnp.where(has, msum / jnp.where(has, ms, 1.0), 0.0)
            tau = jnp.clip(density, 1e-10, None)[..., None]

    pout_ref[...] = p
    pn = _l2n(p)
    fon = fon_ref[...]
    simout_ref[...] = jnp.stack(
        [jax.lax.dot_general(pn[i], fon, dn_last,
                             preferred_element_type=jnp.float32)
         for i in range(G)], axis=0)


def kernel(prototypes, feats, feats_org):
    B, K, D = prototypes.shape
    N = feats.shape[1]
    M = feats_org.shape[0]
    G = _G
    pout, simout = pl.pallas_call(
        _shift_body,
        grid=(B // G,),
        in_specs=[
            pl.BlockSpec((G, K, D), lambda b: (b, 0, 0)),
            pl.BlockSpec((G, N, D), lambda b: (b, 0, 0)),
            pl.BlockSpec((M, D), lambda b: (0, 0)),
        ],
        out_specs=[
            pl.BlockSpec((G, K, D), lambda b: (b, 0, 0)),
            pl.BlockSpec((G, K, M), lambda b: (b, 0, 0)),
        ],
        out_shape=[
            jax.ShapeDtypeStruct((B, K, D), jnp.float32),
            jax.ShapeDtypeStruct((B, K, M), jnp.float32),
        ],
        scratch_shapes=[pltpu.VMEM((M, D), jnp.float32)],
    )(prototypes, feats, feats_org)
    return pout.reshape(B * K, D), simout.reshape(B * K, M)


# speedup vs baseline: 4.1916x; 1.1751x over previous
import jax
import jax.numpy as jnp
from jax.experimental import pallas as pl
from jax.experimental.pallas import tpu as pltpu

_TEMP = 0.1
_TAU0 = 0.1
_NSHIFT = 5


def _l2n(x, eps=1e-8):
    n = jnp.maximum(jnp.sqrt(jnp.sum(x * x, axis=-1, keepdims=True)), eps)
    return x / n


_G = 8


def _shift_body(proto_ref, feats_ref, fo_ref, pout_ref, simout_ref, fon_ref):
    g = pl.program_id(0)
    G, K, D = proto_ref.shape
    N = feats_ref.shape[1]

    @pl.when(g == 0)
    def _():
        fon_ref[...] = _l2n(fo_ref[...])

    f = feats_ref[...]
    fn = _l2n(f)
    p = proto_ref[...]

    dn_last = (((1,), (1,)), ((), ()))
    dn_mid = (((1,), (0,)), ((), ()))

    def sim_all(pn):
        return jnp.stack(
            [jax.lax.dot_general(pn[i], fn[i], dn_last,
                                 preferred_element_type=jnp.float32)
             for i in range(G)], axis=0)

    tau = jnp.full((G, K, 1), _TAU0, jnp.float32)
    sim = sim_all(_l2n(p))
    kiota = jax.lax.broadcasted_iota(jnp.int32, (G, K, N), 1)

    for it in range(_NSHIFT):
        z = sim / (_TEMP * tau)
        z = z - jnp.max(z, axis=2, keepdims=True)
        e = jnp.exp(z)
        w = e / jnp.sum(e, axis=2, keepdims=True)
        colmax = jnp.max(w, axis=1, keepdims=True)
        idx = jnp.min(jnp.where(w == colmax, kiota, K), axis=1, keepdims=True)
        mask = kiota == idx
        w2 = jnp.where(mask, w, 0.0)
        p = jnp.stack(
            [jax.lax.dot_general(w2[i], f[i], dn_mid,
                                 preferred_element_type=jnp.float32)
             for i in range(G)], axis=0)
        if it < _NSHIFT - 1:
            sim = sim_all(_l2n(p))
            msum = jnp.sum(jnp.where(mask, sim, 0.0), axis=2)
            ms = jnp.sum(mask.astype(jnp.float32), axis=2)
            has = ms >= 1.0
            density = 1.0 - jnp.where(has, msum / jnp.where(has, ms, 1.0), 0.0)
            tau = jnp.clip(density, 1e-10, None)[..., None]

    pout_ref[...] = p
    pn = _l2n(p)
    fon = fon_ref[...]
    simout_ref[...] = jnp.stack(
        [jax.lax.dot_general(pn[i], fon, dn_last,
                             preferred_element_type=jnp.float32)
         for i in range(G)], axis=0)


def kernel(prototypes, feats, feats_org):
    B, K, D = prototypes.shape
    N = feats.shape[1]
    M = feats_org.shape[0]
    G = _G
    pout, simout = pl.pallas_call(
        _shift_body,
        grid=(B // G,),
        in_specs=[
            pl.BlockSpec((G, K, D), lambda b: (b, 0, 0)),
            pl.BlockSpec((G, N, D), lambda b: (b, 0, 0)),
            pl.BlockSpec((M, D), lambda b: (0, 0)),
        ],
        out_specs=[
            pl.BlockSpec((G, K, D), lambda b: (b, 0, 0)),
            pl.BlockSpec((G, K, M), lambda b: (b, 0, 0)),
        ],
        out_shape=[
            jax.ShapeDtypeStruct((B, K, D), jnp.float32),
            jax.ShapeDtypeStruct((B, K, M), jnp.float32),
        ],
        scratch_shapes=[pltpu.VMEM((M, D), jnp.float32)],
    )(prototypes, feats, feats_org)
    return pout.reshape(B * K, D), simout.reshape(B * K, M)
